# baseline (speedup 1.0000x reference)
import functools

import jax
import jax.numpy as jnp
from jax import lax
from jax.experimental import pallas as pl
from jax.experimental.pallas import tpu as pltpu
from jax.experimental.pallas import tpu_sc as plsc

_EPS = 1e-12


def _sc_gather_rows(tbl, row_ids):
    _, w = tbl.shape
    bt = row_ids.shape[0]
    nc, ns = 2, 16
    nw = nc * ns
    b_per_w = bt // nw
    ch = 128
    n_ch = b_per_w // ch
    mesh = plsc.VectorSubcoreMesh(core_axis_name="c", subcore_axis_name="s")

    @functools.partial(
        pl.kernel,
        mesh=mesh,
        out_type=jax.ShapeDtypeStruct((bt, w), jnp.float32),
        scratch_types=[
            pltpu.VMEM((b_per_w,), jnp.int32),
            pltpu.VMEM((2, ch, w), jnp.float32),
            pltpu.SemaphoreType.DMA,
            pltpu.SemaphoreType.DMA,
        ],
    )
    def gk(idx_hbm, tbl_hbm, out_hbm, idx_v, bufs, sem0, sem1):
        wid = lax.axis_index("s") * nc + lax.axis_index("c")
        base = wid * b_per_w
        pltpu.sync_copy(idx_hbm.at[pl.ds(base, b_per_w)], idx_v)
        sems = (sem0, sem1)

        def fire(c, slot):
            return pltpu.async_copy(
                tbl_hbm.at[idx_v.at[pl.ds(c * ch, ch)]],
                bufs.at[slot],
                sems[slot],
            )

        def step(c, slot):
            pltpu.make_async_copy(
                tbl_hbm.at[idx_v.at[pl.ds(c * ch, ch)]], bufs.at[slot], sems[slot]
            ).wait()
            pltpu.sync_copy(bufs.at[slot], out_hbm.at[pl.ds(base + c * ch, ch)])
            nxt = c + 2
            @pl.when(nxt < n_ch)
            def _():
                fire(nxt, slot)

        fire(0, 0)
        fire(1, 1)

        def body(c):
            step(2 * c, 0)
            step(2 * c + 1, 1)

        pl.loop(0, n_ch // 2)(body)

    return gk(row_ids, tbl)


def _tc_fused(pad, tok, codebook, tile=2048):
    bt, w = pad.shape
    k_sz, d = codebook.shape
    nt = bt // tile
    ngrp = w // d

    def body(pad_ref, tok_ref, cb_ref, phi_ref, k_ref, commit_ref, perp_ref,
             acc_ref, csum_ref):
        i = pl.program_id(0)
        padv = pad_ref[...]
        grp = tok_ref[...] % ngrp
        ze = jnp.zeros((tile, d), jnp.float32)
        for g in range(ngrp):
            seg = padv[:, g * d:(g + 1) * d]
            ze = jnp.where(grp == g, seg, ze)
        cb = cb_ref[...]
        zn = jnp.sqrt(jnp.sum(ze * ze, axis=1, keepdims=True))
        zf = ze / jnp.maximum(zn, _EPS)
        cn = jnp.sqrt(jnp.sum(cb * cb, axis=1, keepdims=True))
        cbn = cb / jnp.maximum(cn, _EPS)
        sim = lax.dot_general(
            zf, cbn, (((1,), (1,)), ((), ())),
            preferred_element_type=jnp.float32,
            precision=lax.Precision.DEFAULT,
        )
        m = jnp.max(sim, axis=1, keepdims=True)
        col = lax.broadcasted_iota(jnp.int32, sim.shape, 1)
        kk = jnp.min(jnp.where(sim == m, col, k_sz), axis=1)
        k_ref[...] = kk
        e = jnp.exp(sim)
        ones_col = jnp.ones((k_sz, 1), jnp.float32)
        s = lax.dot_general(
            e, ones_col, (((1,), (0,)), ((), ())),
            preferred_element_type=jnp.float32,
        )
        q = e * (1.0 / s)
        ones_row = jnp.ones((1, tile), jnp.float32)
        part = lax.dot_general(
            ones_row, q, (((1,), (0,)), ((), ())),
            preferred_element_type=jnp.float32,
        )
        oh = (col == kk[:, None]).astype(jnp.float32)
        zq = lax.dot_general(
            oh, cb, (((1,), (0,)), ((), ())),
            preferred_element_type=jnp.float32,
            precision=lax.Precision.HIGHEST,
        )
        phi_ref[...] = ze + (zq - ze)
        diff = ze - zq
        cpart = jnp.sum(diff * diff)

        @pl.when(i == 0)
        def _():
            acc_ref[...] = jnp.zeros_like(acc_ref)
            csum_ref[0, 0] = 0.0
            commit_ref[...] = jnp.zeros((1, 1), jnp.float32)
            perp_ref[...] = jnp.zeros((1, 1), jnp.float32)

        acc_ref[...] = acc_ref[...] + part
        csum_ref[0, 0] = csum_ref[0, 0] + cpart

        @pl.when(i == nt - 1)
        def _():
            avg = acc_ref[...] * (1.0 / bt)
            ent = -jnp.sum(avg * jnp.log(avg + 1e-10))
            perp_ref[...] = jnp.exp(ent).reshape(1, 1)
            commit_ref[...] = (csum_ref[0, 0] * (1.0 / (bt * d))).reshape(1, 1)

    return pl.pallas_call(
        body,
        grid=(nt,),
        in_specs=[
            pl.BlockSpec((tile, w), lambda i: (i, 0)),
            pl.BlockSpec((tile, 1), lambda i: (i, 0)),
            pl.BlockSpec((k_sz, d), lambda i: (0, 0)),
        ],
        out_specs=[
            pl.BlockSpec((tile, d), lambda i: (i, 0)),
            pl.BlockSpec((tile,), lambda i: (i,)),
            pl.BlockSpec((1, 1), lambda i: (0, 0)),
            pl.BlockSpec((1, 1), lambda i: (0, 0)),
        ],
        out_shape=[
            jax.ShapeDtypeStruct((bt, d), jnp.float32),
            jax.ShapeDtypeStruct((bt,), jnp.int32),
            jax.ShapeDtypeStruct((1, 1), jnp.float32),
            jax.ShapeDtypeStruct((1, 1), jnp.float32),
        ],
        scratch_shapes=[
            pltpu.VMEM((1, k_sz), jnp.float32),
            pltpu.SMEM((1, 1), jnp.float32),
        ],
    )(pad, tok, codebook)


def kernel(token_ids, embedding, codebook):
    b, t = token_ids.shape
    v, d = embedding.shape
    flat_ids = token_ids.reshape(-1).astype(jnp.int32)
    ngrp = 128 // d
    tbl = embedding.reshape(v // ngrp, 128)
    pad = _sc_gather_rows(tbl, flat_ids // ngrp)
    phi, k, commit, perp = _tc_fused(pad, flat_ids[:, None], codebook)
    return (
        phi.reshape(b, t, -1),
        k.reshape(b, t),
        commit[0, 0],
        perp[0, 0],
    )


# speedup vs baseline: 1.7998x; 1.7998x over previous
import functools

import jax
import jax.numpy as jnp
from jax import lax
from jax.experimental import pallas as pl
from jax.experimental.pallas import tpu as pltpu
from jax.experimental.pallas import tpu_sc as plsc

_EPS = 1e-12


def _sc_gather_rows(tbl, row_ids):
    _, w = tbl.shape
    bt = row_ids.shape[0]
    nc, ns = 2, 16
    nw = nc * ns
    b_per_w = bt // nw
    ch = 128
    n_ch = b_per_w // ch
    mesh = plsc.VectorSubcoreMesh(core_axis_name="c", subcore_axis_name="s")

    @functools.partial(
        pl.kernel,
        mesh=mesh,
        out_type=jax.ShapeDtypeStruct((bt, w), jnp.float32),
        scratch_types=[
            pltpu.VMEM((b_per_w,), jnp.int32),
            pltpu.VMEM((2, ch, w), jnp.float32),
            pltpu.SemaphoreType.DMA,
            pltpu.SemaphoreType.DMA,
        ],
    )
    def gk(idx_hbm, tbl_hbm, out_hbm, idx_v, bufs, sem0, sem1):
        wid = lax.axis_index("s") * nc + lax.axis_index("c")
        base = wid * b_per_w
        pltpu.sync_copy(idx_hbm.at[pl.ds(base, b_per_w)], idx_v)
        sems = (sem0, sem1)

        def fire(c, slot):
            return pltpu.async_copy(
                tbl_hbm.at[idx_v.at[pl.ds(c * ch, ch)]],
                bufs.at[slot],
                sems[slot],
            )

        def step(c, slot):
            pltpu.make_async_copy(
                tbl_hbm.at[idx_v.at[pl.ds(c * ch, ch)]], bufs.at[slot], sems[slot]
            ).wait()
            pltpu.sync_copy(bufs.at[slot], out_hbm.at[pl.ds(base + c * ch, ch)])
            nxt = c + 2
            @pl.when(nxt < n_ch)
            def _():
                fire(nxt, slot)

        fire(0, 0)
        fire(1, 1)

        def body(c):
            step(2 * c, 0)
            step(2 * c + 1, 1)

        pl.loop(0, n_ch // 2)(body)

    return gk(row_ids, tbl)


def _tc_fused(pad, msk, codebook, tile=2048):
    bt, w = pad.shape
    k_sz, d = codebook.shape
    nt = bt // tile
    ngrp = w // d

    def body(pad_ref, msk_ref, cb_ref, phi_ref, k_ref, commit_ref, perp_ref,
             acc_ref, csum_ref):
        i = pl.program_id(0)
        padv = pad_ref[...]
        pm = padv * msk_ref[...].astype(jnp.float32)
        ze = pm[:, 0:d]
        for g in range(1, ngrp):
            ze = ze + pm[:, g * d:(g + 1) * d]
        cb = cb_ref[...]
        zn = jnp.sqrt(jnp.sum(ze * ze, axis=1, keepdims=True))
        zf = ze / jnp.maximum(zn, _EPS)
        cn = jnp.sqrt(jnp.sum(cb * cb, axis=1, keepdims=True))
        cbn = cb / jnp.maximum(cn, _EPS)
        sim = lax.dot_general(
            zf, cbn, (((1,), (1,)), ((), ())),
            preferred_element_type=jnp.float32,
            precision=lax.Precision.DEFAULT,
        )
        m = jnp.max(sim, axis=1, keepdims=True)
        col = lax.broadcasted_iota(jnp.int32, sim.shape, 1)
        kk = jnp.min(jnp.where(sim == m, col, k_sz), axis=1)
        k_ref[...] = kk
        e = jnp.exp(sim)
        s = jnp.sum(e, axis=1, keepdims=True)
        part = jnp.sum(e / s, axis=0)[None, :]
        oh = (col == kk[:, None]).astype(jnp.float32)
        zq = lax.dot_general(
            oh, cb, (((1,), (0,)), ((), ())),
            preferred_element_type=jnp.float32,
            precision=lax.Precision.HIGHEST,
        )
        phi_ref[...] = ze + (zq - ze)
        diff = ze - zq
        cpart = jnp.sum(diff * diff)

        @pl.when(i == 0)
        def _():
            acc_ref[...] = jnp.zeros_like(acc_ref)
            csum_ref[0, 0] = 0.0
            commit_ref[...] = jnp.zeros((1, 1), jnp.float32)
            perp_ref[...] = jnp.zeros((1, 1), jnp.float32)

        acc_ref[...] = acc_ref[...] + part
        csum_ref[0, 0] = csum_ref[0, 0] + cpart

        @pl.when(i == nt - 1)
        def _():
            avg = acc_ref[...] * (1.0 / bt)
            ent = -jnp.sum(avg * jnp.log(avg + 1e-10))
            perp_ref[...] = jnp.exp(ent).reshape(1, 1)
            commit_ref[...] = (csum_ref[0, 0] * (1.0 / (bt * d))).reshape(1, 1)

    return pl.pallas_call(
        body,
        grid=(nt,),
        in_specs=[
            pl.BlockSpec((tile, w), lambda i: (i, 0)),
            pl.BlockSpec((tile, w), lambda i: (i, 0)),
            pl.BlockSpec((k_sz, d), lambda i: (0, 0)),
        ],
        out_specs=[
            pl.BlockSpec((tile, d), lambda i: (i, 0)),
            pl.BlockSpec((tile,), lambda i: (i,)),
            pl.BlockSpec((1, 1), lambda i: (0, 0)),
            pl.BlockSpec((1, 1), lambda i: (0, 0)),
        ],
        out_shape=[
            jax.ShapeDtypeStruct((bt, d), jnp.float32),
            jax.ShapeDtypeStruct((bt,), jnp.int32),
            jax.ShapeDtypeStruct((1, 1), jnp.float32),
            jax.ShapeDtypeStruct((1, 1), jnp.float32),
        ],
        scratch_shapes=[
            pltpu.VMEM((1, k_sz), jnp.float32),
            pltpu.SMEM((1, 1), jnp.float32),
        ],
    )(pad, msk, codebook)


def kernel(token_ids, embedding, codebook):
    b, t = token_ids.shape
    v, d = embedding.shape
    flat_ids = token_ids.reshape(-1).astype(jnp.int32)
    ngrp = 128 // d
    tbl = embedding.reshape(v // ngrp, 128)
    pad = _sc_gather_rows(tbl, flat_ids // ngrp)
    lane_grp = lax.broadcasted_iota(jnp.int32, (b * t, 128), 1) // d
    msk = (lane_grp == (flat_ids % ngrp)[:, None]).astype(jnp.int8)
    phi, k, commit, perp = _tc_fused(pad, msk, codebook)
    return (
        phi.reshape(b, t, -1),
        k.reshape(b, t),
        commit[0, 0],
        perp[0, 0],
    )


# speedup vs baseline: 2.0233x; 1.1242x over previous
import functools

import jax
import jax.numpy as jnp
from jax import lax
from jax.experimental import pallas as pl
from jax.experimental.pallas import tpu as pltpu
from jax.experimental.pallas import tpu_sc as plsc

_EPS = 1e-12


def _sc_gather_rows(tbl, row_ids):
    _, w = tbl.shape
    bt = row_ids.shape[0]
    nc, ns = 2, 16
    nw = nc * ns
    b_per_w = bt // nw
    ch = 128
    n_ch = b_per_w // ch
    mesh = plsc.VectorSubcoreMesh(core_axis_name="c", subcore_axis_name="s")

    @functools.partial(
        pl.kernel,
        mesh=mesh,
        compiler_params=pltpu.CompilerParams(use_tc_tiling_on_sc=False),
        out_type=jax.ShapeDtypeStruct((bt, w), jnp.float32),
        scratch_types=[
            pltpu.VMEM((b_per_w,), jnp.int32),
            pltpu.VMEM((2, ch, w), jnp.float32),
            pltpu.SemaphoreType.DMA,
            pltpu.SemaphoreType.DMA,
        ],
    )
    def gk(idx_hbm, tbl_hbm, out_hbm, idx_v, bufs, sem0, sem1):
        wid = lax.axis_index("s") * nc + lax.axis_index("c")
        base = wid * b_per_w
        pltpu.sync_copy(idx_hbm.at[pl.ds(base, b_per_w)], idx_v)
        sems = (sem0, sem1)

        def fire(c, slot):
            return pltpu.async_copy(
                tbl_hbm.at[idx_v.at[pl.ds(c * ch, ch)]],
                bufs.at[slot],
                sems[slot],
            )

        def step(c, slot):
            pltpu.make_async_copy(
                tbl_hbm.at[idx_v.at[pl.ds(c * ch, ch)]], bufs.at[slot], sems[slot]
            ).wait()
            pltpu.sync_copy(bufs.at[slot], out_hbm.at[pl.ds(base + c * ch, ch)])
            nxt = c + 2
            @pl.when(nxt < n_ch)
            def _():
                fire(nxt, slot)

        fire(0, 0)
        fire(1, 1)

        def body(c):
            step(2 * c, 0)
            step(2 * c + 1, 1)

        pl.loop(0, n_ch // 2)(body)

    return gk(row_ids, tbl)


def _tc_fused(ze, codebook, tile=2048):
    bt, d = ze.shape
    k_sz, _ = codebook.shape
    nt = bt // tile

    def body(pad_ref, cb_ref, phi_ref, k_ref, commit_ref, perp_ref,
             acc_ref, csum_ref):
        i = pl.program_id(0)
        ze = pad_ref[...]
        cb = cb_ref[...]
        zn = jnp.sqrt(jnp.sum(ze * ze, axis=1, keepdims=True))
        zf = ze / jnp.maximum(zn, _EPS)
        cn = jnp.sqrt(jnp.sum(cb * cb, axis=1, keepdims=True))
        cbn = cb / jnp.maximum(cn, _EPS)
        sim = lax.dot_general(
            zf, cbn, (((1,), (1,)), ((), ())),
            preferred_element_type=jnp.float32,
            precision=lax.Precision.DEFAULT,
        )
        m = jnp.max(sim, axis=1, keepdims=True)
        col = lax.broadcasted_iota(jnp.int32, sim.shape, 1)
        kk = jnp.min(jnp.where(sim == m, col, k_sz), axis=1)
        k_ref[...] = kk
        e = jnp.exp(sim)
        s = jnp.sum(e, axis=1, keepdims=True)
        part = jnp.sum(e / s, axis=0)[None, :]
        oh = (col == kk[:, None]).astype(jnp.float32)
        zq = lax.dot_general(
            oh, cb, (((1,), (0,)), ((), ())),
            preferred_element_type=jnp.float32,
            precision=lax.Precision.HIGHEST,
        )
        phi_ref[...] = ze + (zq - ze)
        diff = ze - zq
        cpart = jnp.sum(diff * diff)

        @pl.when(i == 0)
        def _():
            acc_ref[...] = jnp.zeros_like(acc_ref)
            csum_ref[0, 0] = 0.0
            commit_ref[...] = jnp.zeros((1, 1), jnp.float32)
            perp_ref[...] = jnp.zeros((1, 1), jnp.float32)

        acc_ref[...] = acc_ref[...] + part
        csum_ref[0, 0] = csum_ref[0, 0] + cpart

        @pl.when(i == nt - 1)
        def _():
            avg = acc_ref[...] * (1.0 / bt)
            ent = -jnp.sum(avg * jnp.log(avg + 1e-10))
            perp_ref[...] = jnp.exp(ent).reshape(1, 1)
            commit_ref[...] = (csum_ref[0, 0] * (1.0 / (bt * d))).reshape(1, 1)

    return pl.pallas_call(
        body,
        grid=(nt,),
        in_specs=[
            pl.BlockSpec((tile, d), lambda i: (i, 0)),
            pl.BlockSpec((k_sz, d), lambda i: (0, 0)),
        ],
        out_specs=[
            pl.BlockSpec((tile, d), lambda i: (i, 0)),
            pl.BlockSpec((tile,), lambda i: (i,)),
            pl.BlockSpec((1, 1), lambda i: (0, 0)),
            pl.BlockSpec((1, 1), lambda i: (0, 0)),
        ],
        out_shape=[
            jax.ShapeDtypeStruct((bt, d), jnp.float32),
            jax.ShapeDtypeStruct((bt,), jnp.int32),
            jax.ShapeDtypeStruct((1, 1), jnp.float32),
            jax.ShapeDtypeStruct((1, 1), jnp.float32),
        ],
        scratch_shapes=[
            pltpu.VMEM((1, k_sz), jnp.float32),
            pltpu.SMEM((1, 1), jnp.float32),
        ],
    )(ze, codebook)


def kernel(token_ids, embedding, codebook):
    b, t = token_ids.shape
    v, d = embedding.shape
    flat_ids = token_ids.reshape(-1).astype(jnp.int32)
    ze = _sc_gather_rows(embedding, flat_ids)
    phi, k, commit, perp = _tc_fused(ze, codebook)
    return (
        phi.reshape(b, t, -1),
        k.reshape(b, t),
        commit[0, 0],
        perp[0, 0],
    )
